# baseline (speedup 1.0000x reference)
import functools

import jax
import jax.numpy as jnp
from jax import lax
from jax.experimental import pallas as pl
from jax.experimental.pallas import tpu as pltpu
from jax.experimental.pallas import tpu_sc as plsc

N = 10000
D = 128
E = 320000
K = 10
ALPHA = 0.1

NTILES = 16
NCORES = 2
NW = NCORES * NTILES
NP = 10240
RPT = NP // NTILES
CH = 128
GC = 16
CPW = 80
NG = CPW // GC
EP = CPW * NW * CH
DUMP = NP - 1

_mesh = plsc.VectorSubcoreMesh(
    core_axis_name="c", subcore_axis_name="s", num_cores=NCORES)


@functools.partial(
    pl.kernel,
    out_type=jax.ShapeDtypeStruct((NCORES, NP, D), jnp.float32),
    mesh=_mesh,
    scratch_types=[
        pltpu.VMEM_SHARED((NP, D), jnp.float32),
        pltpu.VMEM((2, GC, CH), jnp.int32),
        pltpu.VMEM((2, GC, CH), jnp.int32),
        pltpu.VMEM((2, CH, D), jnp.float32),
        pltpu.SemaphoreType.DMA,
        pltpu.SemaphoreType.DMA,
        pltpu.SemaphoreType.DMA,
    ],
)
def _step_kernel(q_hbm, colp_hbm, rowp_hbm, z_hbm, pp_hbm,
                 pacc, cbufg, rbufg, gbuf, isem, gsem, ssem):
    c = lax.axis_index("c")
    s = lax.axis_index("s")
    w = c * NTILES + s
    r0 = s * RPT

    @pl.when(c == 0)
    def _():
        pltpu.sync_copy(q_hbm.at[pl.ds(r0, RPT)], pacc.at[pl.ds(r0, RPT)])

    @pl.when(c != 0)
    def _():
        pltpu.sync_copy(z_hbm.at[pl.ds(r0, RPT)], pacc.at[pl.ds(r0, RPT)])

    def _load_idx(grp):
        slot = grp % 2
        return (
            pltpu.async_copy(colp_hbm.at[w, pl.ds(grp * GC, GC)],
                             cbufg.at[slot], isem),
            pltpu.async_copy(rowp_hbm.at[w, pl.ds(grp * GC, GC)],
                             rbufg.at[slot], isem),
        )

    ivd = [None] * NG
    ivd[0] = _load_idx(0)
    for dsc in ivd[0]:
        dsc.wait()
    plsc.subcore_barrier()

    def _gather(j):
        grp, k = divmod(j, GC)
        return pltpu.async_copy(q_hbm.at[cbufg.at[grp % 2, k]],
                                gbuf.at[j % 2], gsem)

    gd = [None] * CPW
    sd = [None] * CPW
    gd[0] = _gather(0)
    for j in range(CPW):
        grp, k = divmod(j, GC)
        gd[j].wait()
        if k == 0 and grp + 1 < NG:
            ivd[grp + 1] = _load_idx(grp + 1)
        sd[j] = pltpu.async_copy(gbuf.at[j % 2], pacc.at[rbufg.at[grp % 2, k]],
                                 ssem, add=True)
        if j + 1 < CPW:
            g1, k1 = divmod(j + 1, GC)
            if k1 == 0:
                for dsc in ivd[g1]:
                    dsc.wait()
            if j >= 1:
                sd[j - 1].wait()
            gd[j + 1] = _gather(j + 1)
    sd[CPW - 2].wait()
    sd[CPW - 1].wait()
    plsc.subcore_barrier()
    pltpu.sync_copy(pacc.at[pl.ds(r0, RPT)], pp_hbm.at[c, pl.ds(r0, RPT)])


def _prep_body(x_ref, degw_ref, q0_ref, dinv2_ref, sdeg_ref):
    deg = degw_ref[0, :, 0:1] + degw_ref[1, :, 0:1]
    dinv = lax.rsqrt(deg)
    q0_ref[...] = x_ref[...] * dinv
    dinv2_ref[...] = 1.0 / deg
    sdeg_ref[...] = deg * dinv


_prep = pl.pallas_call(
    _prep_body,
    out_shape=(
        jax.ShapeDtypeStruct((NP, D), jnp.float32),
        jax.ShapeDtypeStruct((NP, 1), jnp.float32),
        jax.ShapeDtypeStruct((NP, 1), jnp.float32),
    ),
)


def _finalize_body(pp_ref, dinv2_ref, q_ref):
    q_ref[...] = (pp_ref[0] + pp_ref[1]) * dinv2_ref[...]


_finalize = pl.pallas_call(
    _finalize_body,
    out_shape=jax.ShapeDtypeStruct((NP, D), jnp.float32),
)

_BR = 1280


def _final_body(x_ref, sdeg_ref, w_ref, b_ref, *qs_out):
    qs, out_ref = qs_out[:-1], qs_out[-1]
    acc = qs[0][...]
    for qr in qs[1:]:
        acc = acc + qr[...]
    t = ((1.0 - ALPHA) / K) * sdeg_ref[...] * acc + ALPHA * x_ref[...]
    out_ref[...] = jnp.dot(t, w_ref[...],
                           preferred_element_type=jnp.float32) + b_ref[...]


_final = pl.pallas_call(
    _final_body,
    grid=(NP // _BR,),
    in_specs=[
        pl.BlockSpec((_BR, D), lambda i: (i, 0)),
        pl.BlockSpec((_BR, 1), lambda i: (i, 0)),
        pl.BlockSpec((D, D), lambda i: (0, 0)),
        pl.BlockSpec((1, D), lambda i: (0, 0)),
    ] + [pl.BlockSpec((_BR, D), lambda i: (i, 0)) for _ in range(K)],
    out_specs=pl.BlockSpec((_BR, D), lambda i: (i, 0)),
    out_shape=jax.ShapeDtypeStruct((NP, D), jnp.float32),
)


@functools.partial(
    pl.kernel,
    out_type=jax.ShapeDtypeStruct((NCORES, NP, D), jnp.float32),
    mesh=_mesh,
    scratch_types=[
        pltpu.VMEM_SHARED((NP, D), jnp.float32),
        pltpu.VMEM((2, GC, CH), jnp.int32),
        pltpu.VMEM((CH, D), jnp.float32),
        pltpu.SemaphoreType.DMA,
        pltpu.SemaphoreType.DMA,
    ],
)
def _deg_kernel(ones_hbm, rowp_hbm, z_hbm, pp_hbm,
                pacc, rbufg, onesb, isem, ssem):
    c = lax.axis_index("c")
    s = lax.axis_index("s")
    w = c * NTILES + s
    r0 = s * RPT

    @pl.when(c == 0)
    def _():
        pltpu.sync_copy(ones_hbm.at[pl.ds(r0, RPT)], pacc.at[pl.ds(r0, RPT)])

    @pl.when(c != 0)
    def _():
        pltpu.sync_copy(z_hbm.at[pl.ds(r0, RPT)], pacc.at[pl.ds(r0, RPT)])

    pltpu.sync_copy(ones_hbm.at[pl.ds(0, CH)], onesb)

    def _load_idx(grp):
        return pltpu.async_copy(rowp_hbm.at[w, pl.ds(grp * GC, GC)],
                                rbufg.at[grp % 2], isem)

    ivd = [None] * NG
    ivd[0] = _load_idx(0)
    ivd[0].wait()
    plsc.subcore_barrier()

    sd = [None] * CPW
    for j in range(CPW):
        grp, k = divmod(j, GC)
        if k == 0 and grp + 1 < NG:
            ivd[grp + 1] = _load_idx(grp + 1)
        if k == GC - 1 and grp + 1 < NG:
            ivd[grp + 1].wait()
        sd[j] = pltpu.async_copy(onesb, pacc.at[rbufg.at[grp % 2, k]],
                                 ssem, add=True)
        if j >= 4:
            sd[j - 4].wait()
    for j in range(CPW - 4, CPW):
        sd[j].wait()
    plsc.subcore_barrier()
    pltpu.sync_copy(pacc.at[pl.ds(r0, RPT)], pp_hbm.at[c, pl.ds(r0, RPT)])


def kernel(x, edge_index, W0, b0):
    x_pad = jnp.pad(x, ((0, NP - N), (0, 0)))
    pad = EP - E
    padi = jnp.arange(pad, dtype=jnp.int32)
    colp = jnp.concatenate(
        [edge_index[1], padi % N]).reshape(NW, CPW, CH)
    rowp = jnp.concatenate(
        [edge_index[0], N + padi % (NP - N)]).reshape(NW, CPW, CH)
    z = jnp.zeros((NP, D), jnp.float32)
    ones = jnp.ones((NP, D), jnp.float32)

    degw = _deg_kernel(ones, rowp, z)
    q, dinv2, sdeg = _prep(x_pad, degw)

    qs = []
    for _ in range(K):
        pp = _step_kernel(q, colp, rowp, z)
        q = _finalize(pp, dinv2)
        qs.append(q)

    out = _final(x_pad, sdeg, W0, b0.reshape(1, D), *qs)
    return out[:N]
